# Initial kernel scaffold; baseline (speedup 1.0000x reference)
import jax
import jax.numpy as jnp
from jax.experimental import pallas as pl


def kernel(in_embs, user_idx, item_idx, gate_w, gate_b):
    raise NotImplementedError("write your pallas kernel here")


# speedup vs baseline: 1.3886x; 1.3886x over previous
import functools

import jax
import jax.numpy as jnp
from jax import lax
from jax.experimental import pallas as pl
from jax.experimental.pallas import tpu as pltpu
from jax.experimental.pallas import tpu_sc as plsc

N_USERS = 30000
N_ITEMS = 20000
N_NODES = N_USERS + N_ITEMS
D = 128
PRUNE = 0.05

R_FIX = 256
NW = 32
T_CHUNKS = 7
C = NW * T_CHUNKS
E_CAP = 5120
G = 64
SLAB_R = R_FIX + 8


def _lane():
    return lax.iota(jnp.int32, 16)


def _norm_body(x_ref, o_ref):
    x = x_ref[...]
    n = jnp.sqrt(jnp.sum(x * x, axis=1, keepdims=True))
    o_ref[...] = x / jnp.maximum(n, 1e-8)


def _normalize(x):
    blk = 200
    return pl.pallas_call(
        _norm_body,
        grid=(N_NODES // blk,),
        in_specs=[pl.BlockSpec((blk, D), lambda i: (i, 0))],
        out_specs=pl.BlockSpec((blk, D), lambda i: (i, 0)),
        out_shape=jax.ShapeDtypeStruct((N_NODES, D), jnp.float32),
    )(x)


def _mean3_body(a_ref, b_ref, c_ref, o_ref):
    o_ref[...] = (a_ref[...] + b_ref[...] + c_ref[...]) * (1.0 / 3.0)


def _mean3(a, b, c):
    blk = 200
    spec = pl.BlockSpec((blk, D), lambda i: (i, 0))
    return pl.pallas_call(
        _mean3_body,
        grid=(N_NODES // blk,),
        in_specs=[spec, spec, spec],
        out_specs=spec,
        out_shape=jax.ShapeDtypeStruct((N_NODES, D), jnp.float32),
    )(a, b, c)


def _sc_layer_kernel():
    mesh = plsc.VectorSubcoreMesh(core_axis_name="c", subcore_axis_name="s")

    @functools.partial(
        pl.kernel,
        mesh=mesh,
        out_type=jax.ShapeDtypeStruct((C * R_FIX, D), jnp.float32),
        compiler_params=pltpu.CompilerParams(needs_layout_passes=False),
        scratch_types=[
            pltpu.VMEM((E_CAP,), jnp.int32),
            pltpu.VMEM((E_CAP,), jnp.int32),
            pltpu.VMEM((E_CAP,), jnp.int32),
            pltpu.VMEM((E_CAP,), jnp.int32),
            pltpu.VMEM((G, D), jnp.float32),
            pltpu.VMEM((G, D), jnp.float32),
            pltpu.VMEM((G, D), jnp.float32),
            pltpu.VMEM((SLAB_R, D), jnp.float32),
            pltpu.VMEM((SLAB_R, 16), jnp.float32),
            pltpu.VMEM((16,), jnp.int32),
            pltpu.VMEM((16,), jnp.float32),
            pltpu.SemaphoreType.DMA,
            pltpu.SemaphoreType.DMA,
            pltpu.SemaphoreType.DMA,
        ],
    )
    def layer(xn_hbm, x_hbm, simrow_hbm, simcol_hbm, col_hbm, rl_hbm,
              meta_hbm, wb_hbm, out_hbm,
              simrow_v, simcol_v, col_v, rl_v,
              rbuf, cbuf, xbuf, acc, deg, meta_v, wb_v,
              sem1, sem2, sem3):
        wid = lax.axis_index("s") * 2 + lax.axis_index("c")
        pltpu.sync_copy(wb_hbm, wb_v)
        wvec = wb_v[...]
        w = jnp.sum(jnp.where(_lane() == 0, wvec, 0.0))
        b = jnp.sum(jnp.where(_lane() == 1, wvec, 0.0))

        def _chunk(t, _carry):
            chunk = wid + t * NW

            def _zero(r, _):
                for k in range(8):
                    acc[r, pl.ds(16 * k, 16)] = jnp.zeros((16,), jnp.float32)
                deg[r, :] = jnp.zeros((16,), jnp.float32)
                return 0

            lax.fori_loop(0, SLAB_R, _zero, 0)

            pltpu.sync_copy(meta_hbm.at[chunk], meta_v)
            pltpu.sync_copy(simrow_hbm.at[chunk], simrow_v)
            pltpu.sync_copy(simcol_hbm.at[chunk], simcol_v)
            pltpu.sync_copy(col_hbm.at[chunk], col_v)
            pltpu.sync_copy(rl_hbm.at[chunk], rl_v)
            mvec = meta_v[...]
            ng = jnp.sum(jnp.where(_lane() == 0, mvec, 0))

            def _group(g, _):
                base = g * G
                cp1 = pltpu.async_copy(
                    xn_hbm.at[simrow_v.at[pl.ds(base, G)]], rbuf, sem1)
                cp2 = pltpu.async_copy(
                    xn_hbm.at[simcol_v.at[pl.ds(base, G)]], cbuf, sem2)
                cp3 = pltpu.async_copy(
                    x_hbm.at[col_v.at[pl.ds(base, G)]], xbuf, sem3)
                cp1.wait()
                cp2.wait()
                cp3.wait()

                def _sub(sg, _):
                    e0 = sg * 16
                    rlv = rl_v[pl.ds(base + e0, 16)]
                    for j in range(16):
                        e = e0 + j
                        rl = jnp.sum(jnp.where(_lane() == j, rlv, 0))
                        dot = rbuf[e, pl.ds(0, 16)] * cbuf[e, pl.ds(0, 16)]
                        for k in range(1, 8):
                            dot = dot + (rbuf[e, pl.ds(16 * k, 16)]
                                         * cbuf[e, pl.ds(16 * k, 16)])
                        s = jnp.sum(dot)
                        sim = (s + 1.0) * 0.5
                        z = sim * w + b
                        zv = jnp.zeros((16,), jnp.float32) + z
                        gate = 1.0 / (1.0 + jnp.exp(-zv))
                        pv = sim * gate
                        pv = jnp.where(pv < PRUNE, 0.0, pv)
                        for k in range(8):
                            plsc.addupdate(acc.at[rl, pl.ds(16 * k, 16)],
                                           pv * xbuf[e, pl.ds(16 * k, 16)])
                        plsc.addupdate(deg.at[rl, :],
                                       jnp.where(_lane() == 0, pv, 0.0))
                    return 0

                lax.fori_loop(0, G // 16, _sub, 0)
                return 0

            lax.fori_loop(0, ng, _group, 0)

            def _flush(r, _):
                d = jnp.sum(deg[r, :])
                dv = jnp.zeros((16,), jnp.float32) + d
                inv = 1.0 / (dv + 1e-7)
                for k in range(8):
                    acc[r, pl.ds(16 * k, 16)] = acc[r, pl.ds(16 * k, 16)] * inv
                return 0

            lax.fori_loop(0, R_FIX, _flush, 0)
            pltpu.sync_copy(acc.at[pl.ds(0, R_FIX)],
                            out_hbm.at[pl.ds(chunk * R_FIX, R_FIX)])
            return 0

        lax.fori_loop(0, T_CHUNKS, _chunk, 0)

    return layer


def kernel(in_embs, user_idx, item_idx, gate_w, gate_b):
    E = user_idx.shape[0]
    user_idx = user_idx.astype(jnp.int32)
    item_idx = item_idx.astype(jnp.int32)

    perm_t = jnp.lexsort((user_idx, item_idx))
    rows_all = jnp.concatenate([user_idx, item_idx[perm_t] + N_USERS])
    cols_all = jnp.concatenate([item_idx + N_USERS, user_idx[perm_t]])
    is_user_row = rows_all < N_USERS
    simrow_idx = jnp.where(is_user_row, rows_all, rows_all + 1)
    simcol_idx = jnp.where(is_user_row, cols_all + 1, cols_all)

    twoE = 2 * E
    starts = jnp.searchsorted(rows_all, jnp.arange(C + 1) * R_FIX)
    cnt = starts[1:] - starts[:-1]
    ngroups = (cnt + (G - 1)) // G
    meta = jnp.zeros((C, 16), jnp.int32).at[:, 0].set(ngroups)

    offs = jnp.arange(E_CAP)
    gidx = starts[:-1][:, None] + offs[None, :]
    valid = offs[None, :] < cnt[:, None]
    gidx_c = jnp.where(valid, jnp.minimum(gidx, twoE - 1), 0)
    simrow_g = jnp.where(valid, simrow_idx[gidx_c], 0).astype(jnp.int32)
    simcol_g = jnp.where(valid, simcol_idx[gidx_c], 0).astype(jnp.int32)
    col_g = jnp.where(valid, cols_all[gidx_c], 0).astype(jnp.int32)
    rl_g = jnp.where(valid,
                     rows_all[gidx_c] - (jnp.arange(C) * R_FIX)[:, None],
                     R_FIX).astype(jnp.int32)

    wb = jnp.zeros((16,), jnp.float32)
    wb = wb.at[0].set(gate_w.reshape(-1)[0]).at[1].set(gate_b.reshape(-1)[0])

    layer = _sc_layer_kernel()

    x0 = in_embs.astype(jnp.float32)
    x = x0
    embs = [x0]
    for _ in range(2):
        xn = _normalize(x)
        y = layer(xn, x, simrow_g, simcol_g, col_g, rl_g, meta, wb)
        x = y[:N_NODES]
        embs.append(x)
    return _mean3(embs[0], embs[1], embs[2])
